# baseline (speedup 1.0000x reference)
import jax
import jax.numpy as jnp
from jax import lax
from jax.experimental import pallas as pl
from jax.experimental.pallas import tpu as pltpu
from jax.experimental.pallas import tpu_sc as plsc

N = 10000
E = 320000
NC = 2
NS = 16
NW = NC * NS

NP = 10016
NSTRIPE = NP // NS

NPD = 10240
NSTRIPED = NPD // NS

L32 = 128
CH32 = 80
EP32 = NW * CH32 * L32

L128 = 120
CH128 = 84
EP128 = NW * CH128 * L128

DEG_G = 8


def _mesh():
    return plsc.VectorSubcoreMesh(
        core_axis_name="c", subcore_axis_name="s", num_cores=NC, num_subcores=NS
    )


def _sc_degrees(srcr, dstr, ones_l, zeros_np):
    def body(src_hbm, dst_hbm, ones_hbm, z_hbm, dsrc_out, ddst_out,
             sidx, didx, ones_v, dsrc_sh, ddst_sh, ssem):
        c = lax.axis_index("c")
        s = lax.axis_index("s")
        w = s * NC + c
        stripe = pl.ds(s * NSTRIPED, NSTRIPED)
        pltpu.sync_copy(z_hbm.at[stripe], dsrc_sh.at[stripe])
        pltpu.sync_copy(z_hbm.at[stripe], ddst_sh.at[stripe])
        pltpu.sync_copy(ones_hbm, ones_v)
        pltpu.sync_copy(src_hbm.at[w], sidx)
        pltpu.sync_copy(dst_hbm.at[w], didx)
        plsc.subcore_barrier()

        def drain_group():
            for _ in range(DEG_G):
                pltpu.make_async_copy(ones_v, dsrc_sh.at[sidx.at[0]], ssem).wait()
                pltpu.make_async_copy(ones_v, ddst_sh.at[didx.at[0]], ssem).wait()

        def group(g, carry):
            @pl.when(g >= 1)
            def _():
                drain_group()
            for b in range(DEG_G):
                j = g * DEG_G + b
                pltpu.async_copy(ones_v, dsrc_sh.at[sidx.at[j]], ssem, add=True)
                pltpu.async_copy(ones_v, ddst_sh.at[didx.at[j]], ssem, add=True)
            return carry

        lax.fori_loop(0, CH32 // DEG_G, group, 0)
        drain_group()
        plsc.subcore_barrier()
        pltpu.sync_copy(dsrc_sh.at[stripe], dsrc_out.at[c, stripe])
        pltpu.sync_copy(ddst_sh.at[stripe], ddst_out.at[c, stripe])

    f = pl.kernel(
        body,
        out_type=(
            jax.ShapeDtypeStruct((NC, NPD), jnp.float32),
            jax.ShapeDtypeStruct((NC, NPD), jnp.float32),
        ),
        mesh=_mesh(),
        scratch_types=[
            pltpu.VMEM((CH32, L32), jnp.int32),
            pltpu.VMEM((CH32, L32), jnp.int32),
            pltpu.VMEM((L32,), jnp.float32),
            pltpu.VMEM_SHARED((NPD,), jnp.float32),
            pltpu.VMEM_SHARED((NPD,), jnp.float32),
            pltpu.SemaphoreType.DMA,
        ],
    )
    return f(srcr, dstr, ones_l, zeros_np)


def _sc_aggregate(h, srcr, dstr, zeros_nd, d, K, L, CH, h_in_spmem=False):
    R = CH // max(K, 1)

    def body(h_hbm, src_hbm, dst_hbm, z_hbm, agg_out, *scratch):
        if h_in_spmem:
            sidx, didx, rows, agg_sh, h_sh, gsem, ssem = scratch
        else:
            sidx, didx, rows, agg_sh, gsem, ssem = scratch
        c = lax.axis_index("c")
        s = lax.axis_index("s")
        w = s * NC + c
        stripe = pl.ds(s * NSTRIPE, NSTRIPE)
        pltpu.sync_copy(z_hbm.at[stripe], agg_sh.at[stripe])
        if h_in_spmem:
            pltpu.sync_copy(h_hbm.at[stripe], h_sh.at[stripe])
        pltpu.sync_copy(src_hbm.at[w], sidx)
        pltpu.sync_copy(dst_hbm.at[w], didx)
        plsc.subcore_barrier()
        h_src = h_sh if h_in_spmem else h_hbm

        if K == 0:
            def chunk(j, carry):
                pltpu.async_copy(h_src.at[sidx.at[j]], rows.at[0], gsem).wait()
                pltpu.sync_copy(rows.at[0], agg_sh.at[didx.at[j]], add=True)
                return carry

            lax.fori_loop(0, CH, chunk, 0)
        else:
            def gather_group(r, grp):
                for b in range(K):
                    pltpu.async_copy(h_src.at[sidx.at[r * K + b]],
                                     rows.at[grp + b], gsem)

            def drain(sem, src_dummy, dst_dummy, count):
                for _ in range(count):
                    pltpu.make_async_copy(src_dummy, dst_dummy, sem).wait()

            gather_group(0, 0)

            def round_body(r, carry):
                p = (r % 2) * K
                q = K - p

                @pl.when(r >= 1)
                def _():
                    drain(ssem, rows.at[0], agg_sh.at[didx.at[0]], K)

                @pl.when(r + 1 < R)
                def _():
                    gather_group(r + 1, q)

                drain(gsem, h_src.at[sidx.at[0]], rows.at[0], K)
                for b in range(K):
                    pltpu.async_copy(rows.at[p + b],
                                     agg_sh.at[didx.at[r * K + b]],
                                     ssem, add=True)
                return carry

            lax.fori_loop(0, R, round_body, 0)
            drain(ssem, rows.at[0], agg_sh.at[didx.at[0]], K)
        plsc.subcore_barrier()
        pltpu.sync_copy(agg_sh.at[stripe], agg_out.at[c, stripe])

    scratch_types = [
        pltpu.VMEM((CH, L), jnp.int32),
        pltpu.VMEM((CH, L), jnp.int32),
        pltpu.VMEM((2 * max(K, 1), L, d), jnp.float32),
        pltpu.VMEM_SHARED((NP, d), jnp.float32),
    ]
    if h_in_spmem:
        scratch_types.append(pltpu.VMEM_SHARED((NP, d), jnp.float32))
    scratch_types += [pltpu.SemaphoreType.DMA, pltpu.SemaphoreType.DMA]

    f = pl.kernel(
        body,
        out_type=jax.ShapeDtypeStruct((NC, NP, d), jnp.float32),
        mesh=_mesh(),
        compiler_params=pltpu.CompilerParams(use_tc_tiling_on_sc=False),
        scratch_types=scratch_types,
    )
    return f(h, srcr, dstr, zeros_nd)


def _tc_norms_h0(degs, degd, z_pad):
    def body(ds_ref, dd_ref, z_ref, ns_ref, nd_ref, h0_ref):
        dsum_s = ds_ref[0] + ds_ref[1]
        dsum_d = dd_ref[0] + dd_ref[1]
        ns = jnp.where(dsum_s > 0, lax.rsqrt(dsum_s), 0.0)
        nd = jnp.where(dsum_d > 0, lax.rsqrt(dsum_d), 0.0)
        ns_ref[...] = ns
        nd_ref[...] = nd
        h0_ref[...] = z_ref[...] * ns

    return pl.pallas_call(
        body,
        out_shape=(
            jax.ShapeDtypeStruct((NP, 1), jnp.float32),
            jax.ShapeDtypeStruct((NP, 1), jnp.float32),
            jax.ShapeDtypeStruct((NP, z_pad.shape[1]), jnp.float32),
        ),
    )(degs, degd, z_pad)


def _tc_layer(agg_part, W, b, norm_dst, norm_src, scale_src, d_out,
              d_in=None, pad_out_to=None):
    d_in = d_in or W.shape[0]
    d_store = pad_out_to or d_out

    def body(a_ref, w_ref, b_ref, nd_ref, ns_ref, o_ref):
        agg = (a_ref[0] + a_ref[1])[:, :d_in]
        r = jnp.dot(agg, w_ref[...], preferred_element_type=jnp.float32)
        r = r * nd_ref[...] + b_ref[...]
        r = jnp.maximum(r, 0.0)
        if scale_src:
            r = r * ns_ref[...]
        if d_store > d_out:
            r = jnp.concatenate(
                [r, jnp.zeros((NP, d_store - d_out), jnp.float32)], axis=1)
        o_ref[...] = r

    return pl.pallas_call(
        body,
        out_shape=jax.ShapeDtypeStruct((NP, d_store), jnp.float32),
    )(agg_part, W, b.reshape(1, -1), norm_dst, norm_src)


def _pad_edges(idx, ep, ch, l):
    padded = jnp.concatenate([idx, jnp.full((ep - E,), N, jnp.int32)])
    return padded.reshape(NW, ch, l)


def kernel(z, edge_index, W1, b1, W2, b2, W3, b3, W4, b4):
    src = edge_index[0]
    dst = edge_index[1]
    srcr = _pad_edges(src, EP32, CH32, L32)
    dstr = _pad_edges(dst, EP32, CH32, L32)
    srcr2 = _pad_edges(src, EP128, CH128, L128)
    dstr2 = _pad_edges(dst, EP128, CH128, L128)
    z_pad = jnp.zeros((NP, z.shape[1]), jnp.float32).at[:N].set(z)

    ones_l = jnp.ones((L32,), jnp.float32)
    zeros_npd = jnp.zeros((NPD,), jnp.float32)

    degs, degd = _sc_degrees(srcr, dstr, ones_l, zeros_npd)
    ns, nd, h0 = _tc_norms_h0(degs[:, :NP, None], degd[:, :NP, None], z_pad)

    zeros32 = jnp.zeros((NP, 32), jnp.float32)
    zeros128 = jnp.zeros((NP, 128), jnp.float32)
    agg = _sc_aggregate(h0, srcr, dstr, zeros32, 32, 8, L32, CH32,
                        h_in_spmem=True)
    h1 = _tc_layer(agg, W1, b1, nd, ns, True, 32)
    agg = _sc_aggregate(h1, srcr, dstr, zeros32, 32, 8, L32, CH32,
                        h_in_spmem=True)
    h2 = _tc_layer(agg, W2, b2, nd, ns, True, 64, pad_out_to=128)
    agg = _sc_aggregate(h2, srcr2, dstr2, zeros128, 128, 0, L128, CH128)
    h3 = _tc_layer(agg, W3, b3, nd, ns, True, 128, d_in=64)
    agg = _sc_aggregate(h3, srcr2, dstr2, zeros128, 128, 0, L128, CH128)
    x4 = _tc_layer(agg, W4, b4, nd, ns, False, 128)
    return x4[:N]


# speedup vs baseline: 1.3969x; 1.3969x over previous
import jax
import jax.numpy as jnp
from jax import lax
from jax.experimental import pallas as pl
from jax.experimental.pallas import tpu as pltpu
from jax.experimental.pallas import tpu_sc as plsc

N = 10000
E = 320000
NC = 2
NS = 16
NW = NC * NS

NP = 10016
NSTRIPE = NP // NS

NPD = 10240
NSTRIPED = NPD // NS

L32 = 128
CH32 = 80
EP32 = NW * CH32 * L32

L128 = 120
CH128 = 84
EP128 = NW * CH128 * L128

DEG_G = 8


def _mesh():
    return plsc.VectorSubcoreMesh(
        core_axis_name="c", subcore_axis_name="s", num_cores=NC, num_subcores=NS
    )


def _sc_degrees(srcr, dstr, ones_l, zeros_np):
    def body(src_hbm, dst_hbm, ones_hbm, z_hbm, dsrc_out, ddst_out,
             sidx, didx, ones_v, dsrc_sh, ddst_sh, ssem):
        c = lax.axis_index("c")
        s = lax.axis_index("s")
        w = s * NC + c
        stripe = pl.ds(s * NSTRIPED, NSTRIPED)
        pltpu.sync_copy(z_hbm.at[stripe], dsrc_sh.at[stripe])
        pltpu.sync_copy(z_hbm.at[stripe], ddst_sh.at[stripe])
        pltpu.sync_copy(ones_hbm, ones_v)
        pltpu.sync_copy(src_hbm.at[w], sidx)
        pltpu.sync_copy(dst_hbm.at[w], didx)
        plsc.subcore_barrier()

        def drain_group():
            for _ in range(DEG_G):
                pltpu.make_async_copy(ones_v, dsrc_sh.at[sidx.at[0]], ssem).wait()
                pltpu.make_async_copy(ones_v, ddst_sh.at[didx.at[0]], ssem).wait()

        def group(g, carry):
            @pl.when(g >= 1)
            def _():
                drain_group()
            for b in range(DEG_G):
                j = g * DEG_G + b
                pltpu.async_copy(ones_v, dsrc_sh.at[sidx.at[j]], ssem, add=True)
                pltpu.async_copy(ones_v, ddst_sh.at[didx.at[j]], ssem, add=True)
            return carry

        lax.fori_loop(0, CH32 // DEG_G, group, 0)
        drain_group()
        plsc.subcore_barrier()
        pltpu.sync_copy(dsrc_sh.at[stripe], dsrc_out.at[c, stripe])
        pltpu.sync_copy(ddst_sh.at[stripe], ddst_out.at[c, stripe])

    f = pl.kernel(
        body,
        out_type=(
            jax.ShapeDtypeStruct((NC, NPD), jnp.float32),
            jax.ShapeDtypeStruct((NC, NPD), jnp.float32),
        ),
        mesh=_mesh(),
        scratch_types=[
            pltpu.VMEM((CH32, L32), jnp.int32),
            pltpu.VMEM((CH32, L32), jnp.int32),
            pltpu.VMEM((L32,), jnp.float32),
            pltpu.VMEM_SHARED((NPD,), jnp.float32),
            pltpu.VMEM_SHARED((NPD,), jnp.float32),
            pltpu.SemaphoreType.DMA,
        ],
    )
    return f(srcr, dstr, ones_l, zeros_np)


def _sc_aggregate(h, srcr, dstr, zeros_nd, d, K, L, CH, h_in_spmem=False):
    R = CH // max(K, 1)

    def body(h_hbm, src_hbm, dst_hbm, z_hbm, agg_out, *scratch):
        if h_in_spmem:
            sidx, didx, rows, agg_sh, h_sh, gsem, ssem = scratch
        else:
            sidx, didx, rows, agg_sh, gsem, ssem = scratch
        c = lax.axis_index("c")
        s = lax.axis_index("s")
        w = s * NC + c
        stripe = pl.ds(s * NSTRIPE, NSTRIPE)
        pltpu.sync_copy(z_hbm.at[stripe], agg_sh.at[stripe])
        if h_in_spmem:
            pltpu.sync_copy(h_hbm.at[stripe], h_sh.at[stripe])
        pltpu.sync_copy(src_hbm.at[w], sidx)
        pltpu.sync_copy(dst_hbm.at[w], didx)
        plsc.subcore_barrier()
        h_src = h_sh if h_in_spmem else h_hbm

        if K == 0:
            def chunk(j, carry):
                pltpu.async_copy(h_src.at[sidx.at[j]], rows.at[0], gsem).wait()
                pltpu.sync_copy(rows.at[0], agg_sh.at[didx.at[j]], add=True)
                return carry

            lax.fori_loop(0, CH, chunk, 0)
        else:
            def gather_group(r, grp):
                for b in range(K):
                    pltpu.async_copy(h_src.at[sidx.at[r * K + b]],
                                     rows.at[grp + b], gsem)

            def drain(sem, src_dummy, dst_dummy, count):
                for _ in range(count):
                    pltpu.make_async_copy(src_dummy, dst_dummy, sem).wait()

            gather_group(0, 0)

            def round_body(r, carry):
                p = (r % 2) * K
                q = K - p

                @pl.when(r >= 1)
                def _():
                    drain(ssem, rows.at[0], agg_sh.at[didx.at[0]], K)

                @pl.when(r + 1 < R)
                def _():
                    gather_group(r + 1, q)

                drain(gsem, h_src.at[sidx.at[0]], rows.at[0], K)
                for b in range(K):
                    pltpu.async_copy(rows.at[p + b],
                                     agg_sh.at[didx.at[r * K + b]],
                                     ssem, add=True)
                return carry

            lax.fori_loop(0, R, round_body, 0)
            drain(ssem, rows.at[0], agg_sh.at[didx.at[0]], K)
        plsc.subcore_barrier()
        pltpu.sync_copy(agg_sh.at[stripe], agg_out.at[c, stripe])

    scratch_types = [
        pltpu.VMEM((CH, L), jnp.int32),
        pltpu.VMEM((CH, L), jnp.int32),
        pltpu.VMEM((2 * max(K, 1), L, d), jnp.float32),
        pltpu.VMEM_SHARED((NP, d), jnp.float32),
    ]
    if h_in_spmem:
        scratch_types.append(pltpu.VMEM_SHARED((NP, d), jnp.float32))
    scratch_types += [pltpu.SemaphoreType.DMA, pltpu.SemaphoreType.DMA]

    f = pl.kernel(
        body,
        out_type=jax.ShapeDtypeStruct((NC, NP, d), jnp.float32),
        mesh=_mesh(),
        compiler_params=pltpu.CompilerParams(use_tc_tiling_on_sc=False),
        scratch_types=scratch_types,
    )
    return f(h, srcr, dstr, zeros_nd)


def _tc_norms_h0(degs, degd, z_pad):
    def body(ds_ref, dd_ref, z_ref, ns_ref, nd_ref, h0_ref):
        dsum_s = ds_ref[0] + ds_ref[1]
        dsum_d = dd_ref[0] + dd_ref[1]
        ns = jnp.where(dsum_s > 0, lax.rsqrt(dsum_s), 0.0)
        nd = jnp.where(dsum_d > 0, lax.rsqrt(dsum_d), 0.0)
        ns_ref[...] = ns
        nd_ref[...] = nd
        h0_ref[...] = z_ref[...] * ns

    return pl.pallas_call(
        body,
        out_shape=(
            jax.ShapeDtypeStruct((NP, 1), jnp.float32),
            jax.ShapeDtypeStruct((NP, 1), jnp.float32),
            jax.ShapeDtypeStruct((NP, z_pad.shape[1]), jnp.float32),
        ),
    )(degs, degd, z_pad)


def _tc_layer(agg_part, W, b, norm_dst, norm_src, scale_src, d_out,
              out_rows=NP):
    def body(a_ref, w_ref, b_ref, nd_ref, ns_ref, o_ref):
        agg = a_ref[0] + a_ref[1]
        r = jnp.dot(agg, w_ref[...], preferred_element_type=jnp.float32)
        r = r * nd_ref[...] + b_ref[...]
        r = jnp.maximum(r, 0.0)
        if scale_src:
            r = r * ns_ref[...]
        o_ref[...] = r[:out_rows]

    return pl.pallas_call(
        body,
        out_shape=jax.ShapeDtypeStruct((out_rows, d_out), jnp.float32),
    )(agg_part, W, b.reshape(1, -1), norm_dst, norm_src)


def _pad_edges(idx, ep, ch, l):
    padded = jnp.concatenate([idx, jnp.full((ep - E,), N, jnp.int32)])
    return padded.reshape(NW, ch, l)


def kernel(z, edge_index, W1, b1, W2, b2, W3, b3, W4, b4):
    src = edge_index[0]
    dst = edge_index[1]
    srcr = _pad_edges(src, EP32, CH32, L32)
    dstr = _pad_edges(dst, EP32, CH32, L32)
    srcr2 = _pad_edges(src, EP128, CH128, L128)
    dstr2 = _pad_edges(dst, EP128, CH128, L128)
    z_pad = jnp.zeros((NP, z.shape[1]), jnp.float32).at[:N].set(z)

    ones_l = jnp.ones((L32,), jnp.float32)
    zeros_npd = jnp.zeros((NPD,), jnp.float32)

    degs, degd = _sc_degrees(srcr, dstr, ones_l, zeros_npd)
    ns, nd, h0 = _tc_norms_h0(degs[:, :NP, None], degd[:, :NP, None], z_pad)

    zeros32 = jnp.zeros((NP, 32), jnp.float32)
    zeros128 = jnp.zeros((NP, 128), jnp.float32)
    agg = _sc_aggregate(h0, srcr, dstr, zeros32, 32, 8, L32, CH32,
                        h_in_spmem=True)
    h1 = _tc_layer(agg, W1, b1, nd, ns, True, 32)
    agg = _sc_aggregate(h1, srcr, dstr, zeros32, 32, 8, L32, CH32,
                        h_in_spmem=True)
    h2 = _tc_layer(agg, W2, b2, nd, ns, True, 64)
    agg = _sc_aggregate(h2, srcr2, dstr2, jnp.zeros((NP, 64), jnp.float32),
                        64, 2, L128, CH128, h_in_spmem=True)
    h3 = _tc_layer(agg, W3, b3, nd, ns, True, 128)
    agg = _sc_aggregate(h3, srcr2, dstr2, zeros128, 128, 1, L128, CH128)
    x4 = _tc_layer(agg, W4, b4, nd, ns, False, 128, out_rows=N)
    return x4


# speedup vs baseline: 1.5464x; 1.1070x over previous
import jax
import jax.numpy as jnp
from jax import lax
from jax.experimental import pallas as pl
from jax.experimental.pallas import tpu as pltpu
from jax.experimental.pallas import tpu_sc as plsc

N = 10000
E = 320000
NC = 2
NS = 16
NW = NC * NS

NP = 10016
NSTRIPE = NP // NS

NPD = 10240
NSTRIPED = NPD // NS

L32 = 128
CH32 = 80
EP32 = NW * CH32 * L32

L128 = 120
CH128 = 84
EP128 = NW * CH128 * L128

DEG_G = 8


def _mesh():
    return plsc.VectorSubcoreMesh(
        core_axis_name="c", subcore_axis_name="s", num_cores=NC, num_subcores=NS
    )


def _sc_degrees(srcr, dstr, ones_l, zeros_np):
    def body(src_hbm, dst_hbm, ones_hbm, z_hbm, dsrc_out, ddst_out,
             sidx, didx, ones_v, dsrc_sh, ddst_sh, ssem):
        c = lax.axis_index("c")
        s = lax.axis_index("s")
        w = s * NC + c
        stripe = pl.ds(s * NSTRIPED, NSTRIPED)
        pltpu.sync_copy(z_hbm.at[stripe], dsrc_sh.at[stripe])
        pltpu.sync_copy(z_hbm.at[stripe], ddst_sh.at[stripe])
        pltpu.sync_copy(ones_hbm, ones_v)
        pltpu.sync_copy(src_hbm.at[w], sidx)
        pltpu.sync_copy(dst_hbm.at[w], didx)
        plsc.subcore_barrier()

        def drain_group():
            for _ in range(DEG_G):
                pltpu.make_async_copy(ones_v, dsrc_sh.at[sidx.at[0]], ssem).wait()
                pltpu.make_async_copy(ones_v, ddst_sh.at[didx.at[0]], ssem).wait()

        def group(g, carry):
            @pl.when(g >= 1)
            def _():
                drain_group()
            for b in range(DEG_G):
                j = g * DEG_G + b
                pltpu.async_copy(ones_v, dsrc_sh.at[sidx.at[j]], ssem, add=True)
                pltpu.async_copy(ones_v, ddst_sh.at[didx.at[j]], ssem, add=True)
            return carry

        lax.fori_loop(0, CH32 // DEG_G, group, 0)
        drain_group()
        plsc.subcore_barrier()
        pltpu.sync_copy(dsrc_sh.at[stripe], dsrc_out.at[c, stripe])
        pltpu.sync_copy(ddst_sh.at[stripe], ddst_out.at[c, stripe])

    f = pl.kernel(
        body,
        out_type=(
            jax.ShapeDtypeStruct((NC, NPD), jnp.float32),
            jax.ShapeDtypeStruct((NC, NPD), jnp.float32),
        ),
        mesh=_mesh(),
        scratch_types=[
            pltpu.VMEM((CH32, L32), jnp.int32),
            pltpu.VMEM((CH32, L32), jnp.int32),
            pltpu.VMEM((L32,), jnp.float32),
            pltpu.VMEM_SHARED((NPD,), jnp.float32),
            pltpu.VMEM_SHARED((NPD,), jnp.float32),
            pltpu.SemaphoreType.DMA,
        ],
    )
    return f(srcr, dstr, ones_l, zeros_np)


def _sc_aggregate(h, srcr, dstr, zeros_nd, d, K, L, CH, h_in_spmem=False):
    R = CH // max(K, 1)

    def body(h_hbm, src_hbm, dst_hbm, z_hbm, agg_out, *scratch):
        if h_in_spmem:
            sidx, didx, rows, agg_sh, h_sh, gsem, ssem = scratch
        else:
            sidx, didx, rows, agg_sh, gsem, ssem = scratch
        c = lax.axis_index("c")
        s = lax.axis_index("s")
        w = s * NC + c
        stripe = pl.ds(s * NSTRIPE, NSTRIPE)
        pltpu.sync_copy(z_hbm.at[stripe], agg_sh.at[stripe])
        if h_in_spmem:
            pltpu.sync_copy(h_hbm.at[stripe], h_sh.at[stripe])
        pltpu.sync_copy(src_hbm.at[w], sidx)
        pltpu.sync_copy(dst_hbm.at[w], didx)
        plsc.subcore_barrier()
        h_src = h_sh if h_in_spmem else h_hbm

        if K == 0:
            def chunk(j, carry):
                pltpu.async_copy(h_src.at[sidx.at[j]], rows.at[0], gsem).wait()
                pltpu.sync_copy(rows.at[0], agg_sh.at[didx.at[j]], add=True)
                return carry

            lax.fori_loop(0, CH, chunk, 0)
        else:
            def gather_group(r, grp):
                for b in range(K):
                    pltpu.async_copy(h_src.at[sidx.at[r * K + b]],
                                     rows.at[grp + b], gsem)

            def drain(sem, src_dummy, dst_dummy, count):
                for _ in range(count):
                    pltpu.make_async_copy(src_dummy, dst_dummy, sem).wait()

            gather_group(0, 0)

            def round_body(r, carry):
                p = (r % 2) * K
                q = K - p

                @pl.when(r >= 1)
                def _():
                    drain(ssem, rows.at[0], agg_sh.at[didx.at[0]], K)

                @pl.when(r + 1 < R)
                def _():
                    gather_group(r + 1, q)

                drain(gsem, h_src.at[sidx.at[0]], rows.at[0], K)
                for b in range(K):
                    pltpu.async_copy(rows.at[p + b],
                                     agg_sh.at[didx.at[r * K + b]],
                                     ssem, add=True)
                return carry

            lax.fori_loop(0, R, round_body, 0)
            drain(ssem, rows.at[0], agg_sh.at[didx.at[0]], K)
        plsc.subcore_barrier()
        pltpu.sync_copy(agg_sh.at[stripe], agg_out.at[c, stripe])

    scratch_types = [
        pltpu.VMEM((CH, L), jnp.int32),
        pltpu.VMEM((CH, L), jnp.int32),
        pltpu.VMEM((2 * max(K, 1), L, d), jnp.float32),
        pltpu.VMEM_SHARED((NP, d), jnp.float32),
    ]
    if h_in_spmem:
        scratch_types.append(pltpu.VMEM_SHARED((NP, d), jnp.float32))
    scratch_types += [pltpu.SemaphoreType.DMA, pltpu.SemaphoreType.DMA]

    f = pl.kernel(
        body,
        out_type=jax.ShapeDtypeStruct((NC, NP, d), jnp.float32),
        mesh=_mesh(),
        compiler_params=pltpu.CompilerParams(use_tc_tiling_on_sc=False),
        scratch_types=scratch_types,
    )
    return f(h, srcr, dstr, zeros_nd)


def _sc_aggregate_split(h_halves, srcr3, dstr3, zeros_nd):
    K, L, CH = 2, L128, CH128
    R = CH // K

    def body(h_hbm, src_hbm, dst_hbm, z_hbm, agg_out,
             sidx, didx, rows, agg_sh, h_sh, gsem, ssem):
        c = lax.axis_index("c")
        s = lax.axis_index("s")
        stripe = pl.ds(s * NSTRIPE, NSTRIPE)
        pltpu.sync_copy(z_hbm.at[stripe], agg_sh.at[stripe])
        pltpu.sync_copy(h_hbm.at[c, stripe], h_sh.at[stripe])
        plsc.subcore_barrier()

        def drain(sem, src_dummy, dst_dummy, count):
            for _ in range(count):
                pltpu.make_async_copy(src_dummy, dst_dummy, sem).wait()

        for ph in range(2):
            pltpu.sync_copy(src_hbm.at[s, ph], sidx)
            pltpu.sync_copy(dst_hbm.at[s, ph], didx)

            def gather_group(r, grp):
                for b in range(K):
                    pltpu.async_copy(h_sh.at[sidx.at[r * K + b]],
                                     rows.at[grp + b], gsem)

            gather_group(0, 0)

            def round_body(r, carry):
                p = (r % 2) * K
                q = K - p

                @pl.when(r >= 1)
                def _():
                    drain(ssem, rows.at[0], agg_sh.at[didx.at[0]], K)

                @pl.when(r + 1 < R)
                def _():
                    gather_group(r + 1, q)

                drain(gsem, h_sh.at[sidx.at[0]], rows.at[0], K)
                for b in range(K):
                    pltpu.async_copy(rows.at[p + b],
                                     agg_sh.at[didx.at[r * K + b]],
                                     ssem, add=True)
                return carry

            lax.fori_loop(0, R, round_body, 0)
            drain(ssem, rows.at[0], agg_sh.at[didx.at[0]], K)

        plsc.subcore_barrier()
        pltpu.sync_copy(agg_sh.at[stripe], agg_out.at[c, stripe])

    f = pl.kernel(
        body,
        out_type=jax.ShapeDtypeStruct((NC, NP, 64), jnp.float32),
        mesh=_mesh(),
        compiler_params=pltpu.CompilerParams(use_tc_tiling_on_sc=False),
        scratch_types=[
            pltpu.VMEM((CH, L), jnp.int32),
            pltpu.VMEM((CH, L), jnp.int32),
            pltpu.VMEM((2 * K, L, 64), jnp.float32),
            pltpu.VMEM_SHARED((NP, 64), jnp.float32),
            pltpu.VMEM_SHARED((NP, 64), jnp.float32),
            pltpu.SemaphoreType.DMA,
            pltpu.SemaphoreType.DMA,
        ],
    )
    return f(h_halves, srcr3, dstr3, zeros_nd)


def _tc_norms_h0(degs, degd, z_pad):
    def body(ds_ref, dd_ref, z_ref, ns_ref, nd_ref, h0_ref):
        dsum_s = ds_ref[0] + ds_ref[1]
        dsum_d = dd_ref[0] + dd_ref[1]
        ns = jnp.where(dsum_s > 0, lax.rsqrt(dsum_s), 0.0)
        nd = jnp.where(dsum_d > 0, lax.rsqrt(dsum_d), 0.0)
        ns_ref[...] = ns
        nd_ref[...] = nd
        h0_ref[...] = z_ref[...] * ns

    return pl.pallas_call(
        body,
        out_shape=(
            jax.ShapeDtypeStruct((NP, 1), jnp.float32),
            jax.ShapeDtypeStruct((NP, 1), jnp.float32),
            jax.ShapeDtypeStruct((NP, z_pad.shape[1]), jnp.float32),
        ),
    )(degs, degd, z_pad)


def _tc_layer(agg_part, W, b, norm_dst, norm_src, scale_src, d_out,
              out_rows=NP):
    def body(a_ref, w_ref, b_ref, nd_ref, ns_ref, o_ref):
        agg = a_ref[0] + a_ref[1]
        r = jnp.dot(agg, w_ref[...], preferred_element_type=jnp.float32)
        r = r * nd_ref[...] + b_ref[...]
        r = jnp.maximum(r, 0.0)
        if scale_src:
            r = r * ns_ref[...]
        o_ref[...] = r[:out_rows]

    return pl.pallas_call(
        body,
        out_shape=jax.ShapeDtypeStruct((out_rows, d_out), jnp.float32),
    )(agg_part, W, b.reshape(1, -1), norm_dst, norm_src)


def _tc_layer3_halves(agg_part, W, b, norm_dst, norm_src):
    def body(a_ref, w_ref, b_ref, nd_ref, ns_ref, o_ref):
        agg = a_ref[0] + a_ref[1]
        r = jnp.dot(agg, w_ref[...], preferred_element_type=jnp.float32)
        r = r * nd_ref[...] + b_ref[...]
        r = jnp.maximum(r, 0.0) * ns_ref[...]
        o_ref[0] = r[:, :64]
        o_ref[1] = r[:, 64:]

    return pl.pallas_call(
        body,
        out_shape=jax.ShapeDtypeStruct((2, NP, 64), jnp.float32),
    )(agg_part, W, b.reshape(1, -1), norm_dst, norm_src)


def _tc_layer4(agg_halves, W, b, norm_dst):
    def body(a_ref, w_ref, b_ref, nd_ref, o_ref):
        ra = jnp.dot(a_ref[0], w_ref[:64, :],
                     preferred_element_type=jnp.float32)
        rb = jnp.dot(a_ref[1], w_ref[64:, :],
                     preferred_element_type=jnp.float32)
        r = (ra + rb) * nd_ref[...] + b_ref[...]
        o_ref[...] = jnp.maximum(r, 0.0)[:N]

    return pl.pallas_call(
        body,
        out_shape=jax.ShapeDtypeStruct((N, 128), jnp.float32),
    )(agg_halves, W, b.reshape(1, -1), norm_dst)


def _pad_edges(idx, ep, ch, l):
    padded = jnp.concatenate([idx, jnp.full((ep - E,), N, jnp.int32)])
    return padded.reshape(NW, ch, l)


def kernel(z, edge_index, W1, b1, W2, b2, W3, b3, W4, b4):
    src = edge_index[0]
    dst = edge_index[1]
    srcr = _pad_edges(src, EP32, CH32, L32)
    dstr = _pad_edges(dst, EP32, CH32, L32)
    srcr2 = _pad_edges(src, EP128, CH128, L128)
    dstr2 = _pad_edges(dst, EP128, CH128, L128)
    srcr3 = srcr2.reshape(NS, 2, CH128, L128)
    dstr3 = dstr2.reshape(NS, 2, CH128, L128)
    z_pad = jnp.zeros((NP, z.shape[1]), jnp.float32).at[:N].set(z)

    ones_l = jnp.ones((L32,), jnp.float32)
    zeros_npd = jnp.zeros((NPD,), jnp.float32)

    degs, degd = _sc_degrees(srcr, dstr, ones_l, zeros_npd)
    ns, nd, h0 = _tc_norms_h0(degs[:, :NP, None], degd[:, :NP, None], z_pad)

    zeros32 = jnp.zeros((NP, 32), jnp.float32)
    zeros128 = jnp.zeros((NP, 128), jnp.float32)
    agg = _sc_aggregate(h0, srcr, dstr, zeros32, 32, 8, L32, CH32,
                        h_in_spmem=True)
    h1 = _tc_layer(agg, W1, b1, nd, ns, True, 32)
    agg = _sc_aggregate(h1, srcr, dstr, zeros32, 32, 8, L32, CH32,
                        h_in_spmem=True)
    zeros64 = jnp.zeros((NP, 64), jnp.float32)
    h2 = _tc_layer(agg, W2, b2, nd, ns, True, 64)
    agg = _sc_aggregate(h2, srcr2, dstr2, zeros64, 64, 2, L128, CH128,
                        h_in_spmem=True)
    h3 = _tc_layer3_halves(agg, W3, b3, nd, ns)
    agg = _sc_aggregate_split(h3, srcr3, dstr3, zeros64)
    x4 = _tc_layer4(agg, W4, b4, nd)
    return x4


# speedup vs baseline: 1.5467x; 1.0002x over previous
import jax
import jax.numpy as jnp
from jax import lax
from jax.experimental import pallas as pl
from jax.experimental.pallas import tpu as pltpu
from jax.experimental.pallas import tpu_sc as plsc

N = 10000
E = 320000
NC = 2
NS = 16
NW = NC * NS

NP = 10016
NSTRIPE = NP // NS

NPD = 10240
NSTRIPED = NPD // NS

L32 = 128
CH32 = 80
EP32 = NW * CH32 * L32

L128 = 120
CH128 = 84
EP128 = NW * CH128 * L128

DEG_G = 8


def _mesh():
    return plsc.VectorSubcoreMesh(
        core_axis_name="c", subcore_axis_name="s", num_cores=NC, num_subcores=NS
    )


def _sc_degrees(srcr, dstr, ones_l, zeros_np):
    def body(src_hbm, dst_hbm, ones_hbm, z_hbm, dsrc_out, ddst_out,
             sidx, didx, ones_v, dsrc_sh, ddst_sh, ssem):
        c = lax.axis_index("c")
        s = lax.axis_index("s")
        w = s * NC + c
        stripe = pl.ds(s * NSTRIPED, NSTRIPED)
        pltpu.sync_copy(z_hbm.at[stripe], dsrc_sh.at[stripe])
        pltpu.sync_copy(z_hbm.at[stripe], ddst_sh.at[stripe])
        pltpu.sync_copy(ones_hbm, ones_v)
        pltpu.sync_copy(src_hbm.at[w], sidx)
        pltpu.sync_copy(dst_hbm.at[w], didx)
        plsc.subcore_barrier()

        def drain_group():
            for _ in range(DEG_G):
                pltpu.make_async_copy(ones_v, dsrc_sh.at[sidx.at[0]], ssem).wait()
                pltpu.make_async_copy(ones_v, ddst_sh.at[didx.at[0]], ssem).wait()

        def group(g, carry):
            @pl.when(g >= 1)
            def _():
                drain_group()
            for b in range(DEG_G):
                j = g * DEG_G + b
                pltpu.async_copy(ones_v, dsrc_sh.at[sidx.at[j]], ssem, add=True)
                pltpu.async_copy(ones_v, ddst_sh.at[didx.at[j]], ssem, add=True)
            return carry

        lax.fori_loop(0, CH32 // DEG_G, group, 0)
        drain_group()
        plsc.subcore_barrier()
        pltpu.sync_copy(dsrc_sh.at[stripe], dsrc_out.at[c, stripe])
        pltpu.sync_copy(ddst_sh.at[stripe], ddst_out.at[c, stripe])

    f = pl.kernel(
        body,
        out_type=(
            jax.ShapeDtypeStruct((NC, NPD), jnp.float32),
            jax.ShapeDtypeStruct((NC, NPD), jnp.float32),
        ),
        mesh=_mesh(),
        scratch_types=[
            pltpu.VMEM((CH32, L32), jnp.int32),
            pltpu.VMEM((CH32, L32), jnp.int32),
            pltpu.VMEM((L32,), jnp.float32),
            pltpu.VMEM_SHARED((NPD,), jnp.float32),
            pltpu.VMEM_SHARED((NPD,), jnp.float32),
            pltpu.SemaphoreType.DMA,
        ],
    )
    return f(srcr, dstr, ones_l, zeros_np)


def _sc_aggregate(h, srcr, dstr, zeros_nd, d, K, L, CH, h_in_spmem=False):
    R = CH // K

    def body(h_hbm, src_hbm, dst_hbm, z_hbm, agg_out, *scratch):
        if h_in_spmem:
            sidx, didx, rows, agg_sh, h_sh, gsem, ssem = scratch
        else:
            sidx, didx, rows, agg_sh, gsem, ssem = scratch
        c = lax.axis_index("c")
        s = lax.axis_index("s")
        w = s * NC + c
        stripe = pl.ds(s * NSTRIPE, NSTRIPE)
        pltpu.sync_copy(z_hbm.at[stripe], agg_sh.at[stripe])
        if h_in_spmem:
            pltpu.sync_copy(h_hbm.at[stripe], h_sh.at[stripe])
        pltpu.sync_copy(src_hbm.at[w], sidx)
        pltpu.sync_copy(dst_hbm.at[w], didx)
        plsc.subcore_barrier()
        h_src = h_sh if h_in_spmem else h_hbm

        def gather_group(r, grp):
            for b in range(K):
                pltpu.async_copy(h_src.at[sidx.at[r * K + b]],
                                 rows.at[grp + b], gsem)

        def drain(sem, src_dummy, dst_dummy, count):
            for _ in range(count):
                pltpu.make_async_copy(src_dummy, dst_dummy, sem).wait()

        gather_group(0, 0)

        def round_body(r, carry):
            p = (r % 2) * K
            q = K - p

            @pl.when(r >= 1)
            def _():
                drain(ssem, rows.at[0], agg_sh.at[didx.at[0]], K)

            @pl.when(r + 1 < R)
            def _():
                gather_group(r + 1, q)

            drain(gsem, h_src.at[sidx.at[0]], rows.at[0], K)
            for b in range(K):
                pltpu.async_copy(rows.at[p + b],
                                 agg_sh.at[didx.at[r * K + b]],
                                 ssem, add=True)
            return carry

        lax.fori_loop(0, R, round_body, 0)
        drain(ssem, rows.at[0], agg_sh.at[didx.at[0]], K)
        plsc.subcore_barrier()
        pltpu.sync_copy(agg_sh.at[stripe], agg_out.at[c, stripe])

    scratch_types = [
        pltpu.VMEM((CH, L), jnp.int32),
        pltpu.VMEM((CH, L), jnp.int32),
        pltpu.VMEM((2 * K, L, d), jnp.float32),
        pltpu.VMEM_SHARED((NP, d), jnp.float32),
    ]
    if h_in_spmem:
        scratch_types.append(pltpu.VMEM_SHARED((NP, d), jnp.float32))
    scratch_types += [pltpu.SemaphoreType.DMA, pltpu.SemaphoreType.DMA]

    f = pl.kernel(
        body,
        out_type=jax.ShapeDtypeStruct((NC, NP, d), jnp.float32),
        mesh=_mesh(),
        compiler_params=pltpu.CompilerParams(use_tc_tiling_on_sc=False),
        scratch_types=scratch_types,
    )
    return f(h, srcr, dstr, zeros_nd)


def _sc_aggregate_split(h_halves, srcr3, dstr3, zeros_nd):
    K, L, CH = 2, L128, CH128
    R = CH // K

    def body(h_hbm, src_hbm, dst_hbm, z_hbm, agg_out,
             sidx, didx, rows, agg_sh, h_sh, gsem, ssem):
        c = lax.axis_index("c")
        s = lax.axis_index("s")
        stripe = pl.ds(s * NSTRIPE, NSTRIPE)
        pltpu.sync_copy(z_hbm.at[stripe], agg_sh.at[stripe])
        pltpu.sync_copy(h_hbm.at[c, stripe], h_sh.at[stripe])
        plsc.subcore_barrier()

        def drain(sem, src_dummy, dst_dummy, count):
            for _ in range(count):
                pltpu.make_async_copy(src_dummy, dst_dummy, sem).wait()

        for ph in range(2):
            pltpu.sync_copy(src_hbm.at[s, ph], sidx)
            pltpu.sync_copy(dst_hbm.at[s, ph], didx)

            def gather_group(r, grp):
                for b in range(K):
                    pltpu.async_copy(h_sh.at[sidx.at[r * K + b]],
                                     rows.at[grp + b], gsem)

            gather_group(0, 0)

            def round_body(r, carry):
                p = (r % 2) * K
                q = K - p

                @pl.when(r >= 1)
                def _():
                    drain(ssem, rows.at[0], agg_sh.at[didx.at[0]], K)

                @pl.when(r + 1 < R)
                def _():
                    gather_group(r + 1, q)

                drain(gsem, h_sh.at[sidx.at[0]], rows.at[0], K)
                for b in range(K):
                    pltpu.async_copy(rows.at[p + b],
                                     agg_sh.at[didx.at[r * K + b]],
                                     ssem, add=True)
                return carry

            lax.fori_loop(0, R, round_body, 0)
            drain(ssem, rows.at[0], agg_sh.at[didx.at[0]], K)

        plsc.subcore_barrier()
        pltpu.sync_copy(agg_sh.at[stripe], agg_out.at[c, stripe])

    f = pl.kernel(
        body,
        out_type=jax.ShapeDtypeStruct((NC, NP, 64), jnp.float32),
        mesh=_mesh(),
        compiler_params=pltpu.CompilerParams(use_tc_tiling_on_sc=False),
        scratch_types=[
            pltpu.VMEM((CH, L), jnp.int32),
            pltpu.VMEM((CH, L), jnp.int32),
            pltpu.VMEM((2 * K, L, 64), jnp.float32),
            pltpu.VMEM_SHARED((NP, 64), jnp.float32),
            pltpu.VMEM_SHARED((NP, 64), jnp.float32),
            pltpu.SemaphoreType.DMA,
            pltpu.SemaphoreType.DMA,
        ],
    )
    return f(h_halves, srcr3, dstr3, zeros_nd)


def _tc_norms_h0(degs, degd, z_pad):
    def body(ds_ref, dd_ref, z_ref, ns_ref, nd_ref, h0_ref):
        dsum_s = ds_ref[0] + ds_ref[1]
        dsum_d = dd_ref[0] + dd_ref[1]
        ns = jnp.where(dsum_s > 0, lax.rsqrt(dsum_s), 0.0)
        nd = jnp.where(dsum_d > 0, lax.rsqrt(dsum_d), 0.0)
        ns_ref[...] = ns
        nd_ref[...] = nd
        h0_ref[...] = z_ref[...] * ns

    return pl.pallas_call(
        body,
        out_shape=(
            jax.ShapeDtypeStruct((NP, 1), jnp.float32),
            jax.ShapeDtypeStruct((NP, 1), jnp.float32),
            jax.ShapeDtypeStruct((NP, z_pad.shape[1]), jnp.float32),
        ),
    )(degs, degd, z_pad)


def _tc_layer(agg_part, W, b, norm_dst, norm_src, scale_src, d_out,
              out_rows=NP):
    def body(a_ref, w_ref, b_ref, nd_ref, ns_ref, o_ref):
        agg = a_ref[0] + a_ref[1]
        r = jnp.dot(agg, w_ref[...], preferred_element_type=jnp.float32)
        r = r * nd_ref[...] + b_ref[...]
        r = jnp.maximum(r, 0.0)
        if scale_src:
            r = r * ns_ref[...]
        o_ref[...] = r[:out_rows]

    return pl.pallas_call(
        body,
        out_shape=jax.ShapeDtypeStruct((out_rows, d_out), jnp.float32),
    )(agg_part, W, b.reshape(1, -1), norm_dst, norm_src)


def _tc_layer3_halves(agg_part, W, b, norm_dst, norm_src):
    def body(a_ref, w_ref, b_ref, nd_ref, ns_ref, o_ref):
        agg = a_ref[0] + a_ref[1]
        r = jnp.dot(agg, w_ref[...], preferred_element_type=jnp.float32)
        r = r * nd_ref[...] + b_ref[...]
        r = jnp.maximum(r, 0.0) * ns_ref[...]
        o_ref[0] = r[:, :64]
        o_ref[1] = r[:, 64:]

    return pl.pallas_call(
        body,
        out_shape=jax.ShapeDtypeStruct((2, NP, 64), jnp.float32),
    )(agg_part, W, b.reshape(1, -1), norm_dst, norm_src)


def _tc_layer4(agg_halves, W, b, norm_dst):
    def body(a_ref, w_ref, b_ref, nd_ref, o_ref):
        ra = jnp.dot(a_ref[0], w_ref[:64, :],
                     preferred_element_type=jnp.float32)
        rb = jnp.dot(a_ref[1], w_ref[64:, :],
                     preferred_element_type=jnp.float32)
        r = (ra + rb) * nd_ref[...] + b_ref[...]
        o_ref[...] = jnp.maximum(r, 0.0)[:N]

    return pl.pallas_call(
        body,
        out_shape=jax.ShapeDtypeStruct((N, 128), jnp.float32),
    )(agg_halves, W, b.reshape(1, -1), norm_dst)


def _pad_edges(idx, ep, ch, l):
    padded = jnp.concatenate([idx, jnp.full((ep - E,), N, jnp.int32)])
    return padded.reshape(NW, ch, l)


def kernel(z, edge_index, W1, b1, W2, b2, W3, b3, W4, b4):
    src = edge_index[0]
    dst = edge_index[1]
    srcr = _pad_edges(src, EP32, CH32, L32)
    dstr = _pad_edges(dst, EP32, CH32, L32)
    srcr2 = _pad_edges(src, EP128, CH128, L128)
    dstr2 = _pad_edges(dst, EP128, CH128, L128)
    srcr3 = srcr2.reshape(NS, 2, CH128, L128)
    dstr3 = dstr2.reshape(NS, 2, CH128, L128)
    z_pad = jnp.zeros((NP, z.shape[1]), jnp.float32).at[:N].set(z)

    ones_l = jnp.ones((L32,), jnp.float32)
    zeros_npd = jnp.zeros((NPD,), jnp.float32)

    degs, degd = _sc_degrees(srcr, dstr, ones_l, zeros_npd)
    ns, nd, h0 = _tc_norms_h0(degs[:, :NP, None], degd[:, :NP, None], z_pad)

    zeros32 = jnp.zeros((NP, 32), jnp.float32)
    agg = _sc_aggregate(h0, srcr, dstr, zeros32, 32, 8, L32, CH32,
                        h_in_spmem=True)
    h1 = _tc_layer(agg, W1, b1, nd, ns, True, 32)
    agg = _sc_aggregate(h1, srcr, dstr, zeros32, 32, 8, L32, CH32,
                        h_in_spmem=True)
    zeros64 = jnp.zeros((NP, 64), jnp.float32)
    h2 = _tc_layer(agg, W2, b2, nd, ns, True, 64)
    agg = _sc_aggregate(h2, srcr2, dstr2, zeros64, 64, 2, L128, CH128,
                        h_in_spmem=True)
    h3 = _tc_layer3_halves(agg, W3, b3, nd, ns)
    agg = _sc_aggregate_split(h3, srcr3, dstr3, zeros64)
    x4 = _tc_layer4(agg, W4, b4, nd)
    return x4
